# baseline (speedup 1.0000x reference)
import functools

import jax
import jax.numpy as jnp
from jax import lax
from jax.experimental import pallas as pl
from jax.experimental.pallas import tpu as pltpu
from jax.experimental.pallas import tpu_sc as plsc

_N = 10000
_E = 320000
_D = 128

_NTILE = 32
_CHUNK = 128
_GRP = 8
_NGRP = -(-_E // (_NTILE * _CHUNK * _GRP))
_NCHUNK = _NGRP * _GRP
_EPAD = _NTILE * _NCHUNK * _CHUNK
_ROWS_SH = 10240
_ZROWS = _ROWS_SH // 16

_BLK = 1000
_GRID = _N // _BLK


def _gelu(x):
    return x * 0.5 * (1.0 + lax.erf(x * 0.7071067811865476))


def _dot(a, b):
    return jnp.dot(a, b, preferred_element_type=jnp.float32)


_SC_MESH = plsc.VectorSubcoreMesh(core_axis_name="c", subcore_axis_name="s")


def _make_agg():
    scratch = [
        pltpu.VMEM((_GRP, _CHUNK), jnp.int32),
        pltpu.VMEM((_GRP, _CHUNK), jnp.int32),
        pltpu.VMEM((_GRP, _CHUNK), jnp.int32),
        pltpu.VMEM((_GRP, _CHUNK), jnp.int32),
        pltpu.VMEM((_CHUNK, _D), jnp.float32),
        pltpu.VMEM((_CHUNK, _D), jnp.float32),
        pltpu.VMEM_SHARED((_ROWS_SH, _D), jnp.float32),
        pltpu.SemaphoreType.DMA,
        pltpu.SemaphoreType.DMA,
        pltpu.SemaphoreType.DMA,
        pltpu.SemaphoreType.DMA,
    ]

    def body(src3, dst3, z128, m_hbm, m2_hbm, out_p,
             src0, dst0, src1, dst1, rows0, rows1, acc_sh,
             gs0, gs1, is0, is1):
        cid = lax.axis_index("c")
        sid = lax.axis_index("s")
        wid = cid * 16 + sid
        rows = (rows0, rows1)
        gsem = (gs0, gs1)
        pltpu.sync_copy(z128, rows0)
        for z in range(_ZROWS // 128):
            pltpu.sync_copy(rows0, acc_sh.at[pl.ds(sid * _ZROWS + z * 128, 128)])
        plsc.subcore_barrier()

        def load_idx(g, sv, dv, sem):
            pltpu.async_copy(src3.at[wid, pl.ds(g * _GRP, _GRP)], sv, sem)
            pltpu.async_copy(dst3.at[wid, pl.ds(g * _GRP, _GRP)], dv, sem)

        def wait_idx(sv, dv, sem):
            pltpu.make_async_copy(src3.at[wid, pl.ds(0, _GRP)], sv, sem).wait()
            pltpu.make_async_copy(dst3.at[wid, pl.ds(0, _GRP)], dv, sem).wait()

        def run_group(mref, sv, dv):
            cp = pltpu.async_copy(mref.at[sv.at[0]], rows[0], gsem[0])
            for b in range(_GRP):
                p = b % 2
                if b + 1 < _GRP:
                    nxt = pltpu.async_copy(mref.at[sv.at[b + 1]],
                                           rows[1 - p], gsem[1 - p])
                cp.wait()
                pltpu.sync_copy(rows[p], acc_sh.at[dv.at[b]], add=True)
                if b + 1 < _GRP:
                    cp = nxt

        load_idx(0, src0, dst0, is0)
        def make_super_step(mref):
            def super_step(s, carry):
                wait_idx(src0, dst0, is0)
                load_idx(2 * s + 1, src1, dst1, is1)
                run_group(mref, src0, dst0)
                wait_idx(src1, dst1, is1)
                load_idx(2 * s + 2, src0, dst0, is0)
                run_group(mref, src1, dst1)
                return carry
            return super_step

        @pl.when(cid == 0)
        def _():
            lax.fori_loop(0, _NGRP // 2, make_super_step(m_hbm), 0)

        @pl.when(cid == 1)
        def _():
            lax.fori_loop(0, _NGRP // 2, make_super_step(m2_hbm), 0)
        wait_idx(src0, dst0, is0)
        plsc.subcore_barrier()
        for z in range(_ZROWS // 128):
            r0 = sid * _ZROWS + z * 128
            pltpu.sync_copy(acc_sh.at[pl.ds(r0, 128)], rows0)
            pltpu.sync_copy(rows0, out_p.at[cid, pl.ds(r0, 128)])

    return pl.kernel(body,
                     out_type=(jax.ShapeDtypeStruct((2, _ROWS_SH, _D),
                                                    jnp.float32),),
                     mesh=_SC_MESH, scratch_types=scratch)


def _make_deg():
    scratch = [
        pltpu.VMEM((_GRP, _CHUNK), jnp.int32),
        pltpu.VMEM((_CHUNK, _D), jnp.float32),
        pltpu.VMEM_SHARED((_ROWS_SH, _D), jnp.float32),
        pltpu.SemaphoreType.DMA,
    ]

    def body(dst3, z128, ones128, deg_p, dst_v, buf_v, deg_sh, sem):
        cid = lax.axis_index("c")
        sid = lax.axis_index("s")
        wid = cid * 16 + sid
        pltpu.sync_copy(z128, buf_v)
        for z in range(_ZROWS // 128):
            pltpu.sync_copy(buf_v, deg_sh.at[pl.ds(sid * _ZROWS + z * 128, 128)])
        pltpu.sync_copy(ones128, buf_v)
        plsc.subcore_barrier()

        def group(g, carry):
            pltpu.sync_copy(dst3.at[wid, pl.ds(g * _GRP, _GRP)], dst_v)
            for j in range(_GRP):
                pltpu.sync_copy(buf_v, deg_sh.at[dst_v.at[j]], add=True)
            return carry

        lax.fori_loop(0, _NGRP, group, 0)
        plsc.subcore_barrier()
        for z in range(_ZROWS // 128):
            r0 = sid * _ZROWS + z * 128
            pltpu.sync_copy(deg_sh.at[pl.ds(r0, 128)], buf_v)
            pltpu.sync_copy(buf_v, deg_p.at[cid, pl.ds(r0, 128)])

    return pl.kernel(body,
                     out_type=(jax.ShapeDtypeStruct((2, _ROWS_SH, _D),
                                                    jnp.float32),),
                     mesh=_SC_MESH, scratch_types=scratch)


_agg = _make_agg()
_deg = _make_deg()


def _wspec():
    return pl.BlockSpec((_D, _D), lambda i: (0, 0))


def _bspec():
    return pl.BlockSpec((_D,), lambda i: (0,))


def _hspec():
    return pl.BlockSpec((_BLK, _D), lambda i: (i, 0))


def _stage_in(feats, W_in, b_in, W1, b1):
    def body(f, win, bin_, w1, b1_, h_o, m_o, m2_o):
        h = _gelu(_dot(f[...], win[...]) + bin_[...])
        h_o[...] = h
        m = _gelu(_dot(h, w1[...]) + b1_[...])
        m_o[...] = m
        m2_o[...] = m

    return pl.pallas_call(
        body,
        grid=(_GRID,),
        in_specs=[_hspec(), _wspec(), _bspec(), _wspec(), _bspec()],
        out_specs=[_hspec(), _hspec(), _hspec()],
        out_shape=[jax.ShapeDtypeStruct((_N, _D), jnp.float32)] * 3,
    )(feats, W_in, b_in, W1, b1)


_PSPEC = pl.BlockSpec((2, _BLK, _D), lambda i: (0, i, 0))
_DSPEC = pl.BlockSpec((2, _BLK, _D), lambda i: (0, i, 0))


def _stage_mid(h, p, dp, W2, b2, W3, b3, Wl, bl, W1n, b1n):
    def body(h_r, p_r, d_r, w2, b2_, w3, b3_, wl, bl_, w1n, b1n_,
             h_o, m_o, m2_o):
        d = d_r[0, :, :1] + d_r[1, :, :1]
        agg = (p_r[0] + p_r[1]) * (1.0 / jnp.maximum(d, 1.0))
        h0 = h_r[...]
        t = _gelu(_dot(agg, w2[...]) + b2_[...] + _dot(h0, w3[...]) + b3_[...])
        hn = _dot(t, wl[...]) + bl_[...] + h0
        h_o[...] = hn
        m = _gelu(_dot(hn, w1n[...]) + b1n_[...])
        m_o[...] = m
        m2_o[...] = m

    return pl.pallas_call(
        body,
        grid=(_GRID,),
        in_specs=[_hspec(), _PSPEC, _DSPEC, _wspec(), _bspec(), _wspec(),
                  _bspec(), _wspec(), _bspec(), _wspec(), _bspec()],
        out_specs=[_hspec(), _hspec(), _hspec()],
        out_shape=[jax.ShapeDtypeStruct((_N, _D), jnp.float32)] * 3,
    )(h, p, dp, W2, b2, W3, b3, Wl, bl, W1n, b1n)


def _stage_out(h, p, dp, W2, b2, W3, b3, Wl, bl, W_out, b_out):
    def body(h_r, p_r, d_r, w2, b2_, w3, b3_, wl, bl_, wo, bo_, o_o):
        d = d_r[0, :, :1] + d_r[1, :, :1]
        agg = (p_r[0] + p_r[1]) * (1.0 / jnp.maximum(d, 1.0))
        h0 = h_r[...]
        t = _gelu(_dot(agg, w2[...]) + b2_[...] + _dot(h0, w3[...]) + b3_[...])
        hn = _dot(t, wl[...]) + bl_[...] + h0
        o_o[...] = _dot(hn, wo[...]) + bo_[...]

    return pl.pallas_call(
        body,
        grid=(_GRID,),
        in_specs=[_hspec(), _PSPEC, _DSPEC, _wspec(), _bspec(), _wspec(),
                  _bspec(), _wspec(), _bspec(), _wspec(), _bspec()],
        out_specs=_hspec(),
        out_shape=jax.ShapeDtypeStruct((_N, _D), jnp.float32),
    )(h, p, dp, W2, b2, W3, b3, Wl, bl, W_out, b_out)


def kernel(feats, edge_index, W_in, b_in,
           W1_0, b1_0, W2_0, b2_0, W3_0, b3_0, Wl_0, bl_0,
           W1_1, b1_1, W2_1, b2_1, W3_1, b3_1, Wl_1, bl_1,
           W_out, b_out):
    src = edge_index[0]
    dst = edge_index[1]
    pad = _EPAD - _E
    src3 = jnp.concatenate(
        [src, jnp.zeros((pad,), jnp.int32)]).reshape(_NTILE, _NCHUNK, _CHUNK)
    src3 = jnp.concatenate(
        [src3, jnp.zeros((_NTILE, _GRP, _CHUNK), jnp.int32)], axis=1)
    dst3 = jnp.concatenate(
        [dst, jnp.full((pad,), _N, jnp.int32)]).reshape(_NTILE, _NCHUNK, _CHUNK)
    dst3 = jnp.concatenate(
        [dst3, jnp.full((_NTILE, _GRP, _CHUNK), _N, jnp.int32)], axis=1)
    z128 = jnp.zeros((128, _D), jnp.float32)
    ones128 = jnp.ones((_CHUNK, _D), jnp.float32)

    (dp,) = _deg(dst3, z128, ones128)
    h, m, m2 = _stage_in(feats, W_in, b_in, W1_0, b1_0)
    (p0,) = _agg(src3, dst3, z128, m, m2)
    h, m, m2 = _stage_mid(h, p0, dp, W2_0, b2_0, W3_0, b3_0, Wl_0, bl_0,
                          W1_1, b1_1)
    (p1,) = _agg(src3, dst3, z128, m, m2)
    out = _stage_out(h, p1, dp, W2_1, b2_1, W3_1, b3_1, Wl_1, bl_1,
                     W_out, b_out)
    return out


# speedup vs baseline: 1.0164x; 1.0164x over previous
import functools

import jax
import jax.numpy as jnp
from jax import lax
from jax.experimental import pallas as pl
from jax.experimental.pallas import tpu as pltpu
from jax.experimental.pallas import tpu_sc as plsc

_N = 10000
_E = 320000
_D = 128

_NTILE = 32
_CHUNK = 128
_GRP = 8
_NGRP = -(-_E // (_NTILE * _CHUNK * _GRP))
_NCHUNK = _NGRP * _GRP
_EPAD = _NTILE * _NCHUNK * _CHUNK
_ROWS_SH = 10240
_ZROWS = _ROWS_SH // 16

_BLK = 1000
_GRID = _N // _BLK


def _gelu(x):
    return x * 0.5 * (1.0 + lax.erf(x * 0.7071067811865476))


def _dot(a, b):
    return jnp.dot(a, b, preferred_element_type=jnp.float32)


_SC_MESH = plsc.VectorSubcoreMesh(core_axis_name="c", subcore_axis_name="s")


def _make_agg():
    scratch = [
        pltpu.VMEM((_GRP, _CHUNK), jnp.int32),
        pltpu.VMEM((_GRP, _CHUNK), jnp.int32),
        pltpu.VMEM((_GRP, _CHUNK), jnp.int32),
        pltpu.VMEM((_GRP, _CHUNK), jnp.int32),
        pltpu.VMEM((_CHUNK, _D), jnp.float32),
        pltpu.VMEM((_CHUNK, _D), jnp.float32),
        pltpu.VMEM_SHARED((_ROWS_SH, _D), jnp.float32),
        pltpu.SemaphoreType.DMA,
        pltpu.SemaphoreType.DMA,
        pltpu.SemaphoreType.DMA,
        pltpu.SemaphoreType.DMA,
    ]

    def body(src3, dst3, z128, m_hbm, out_p,
             src0, dst0, src1, dst1, rows0, rows1, acc_sh,
             gs0, gs1, is0, is1):
        cid = lax.axis_index("c")
        sid = lax.axis_index("s")
        wid = cid * 16 + sid
        rows = (rows0, rows1)
        gsem = (gs0, gs1)
        pltpu.sync_copy(z128, rows0)
        for z in range(_ZROWS // 128):
            pltpu.sync_copy(rows0, acc_sh.at[pl.ds(sid * _ZROWS + z * 128, 128)])
        plsc.subcore_barrier()

        def load_idx(g, sv, dv, sem):
            pltpu.async_copy(src3.at[wid, pl.ds(g * _GRP, _GRP)], sv, sem)
            pltpu.async_copy(dst3.at[wid, pl.ds(g * _GRP, _GRP)], dv, sem)

        def wait_idx(sv, dv, sem):
            pltpu.make_async_copy(src3.at[wid, pl.ds(0, _GRP)], sv, sem).wait()
            pltpu.make_async_copy(dst3.at[wid, pl.ds(0, _GRP)], dv, sem).wait()

        def run_group(sv, dv):
            cp = pltpu.async_copy(m_hbm.at[sv.at[0]], rows[0], gsem[0])
            for b in range(_GRP):
                p = b % 2
                if b + 1 < _GRP:
                    nxt = pltpu.async_copy(m_hbm.at[sv.at[b + 1]],
                                           rows[1 - p], gsem[1 - p])
                cp.wait()
                pltpu.sync_copy(rows[p], acc_sh.at[dv.at[b]], add=True)
                if b + 1 < _GRP:
                    cp = nxt

        load_idx(0, src0, dst0, is0)
        def super_step(s, carry):
            wait_idx(src0, dst0, is0)
            load_idx(2 * s + 1, src1, dst1, is1)
            run_group(src0, dst0)
            wait_idx(src1, dst1, is1)
            load_idx(2 * s + 2, src0, dst0, is0)
            run_group(src1, dst1)
            return carry

        lax.fori_loop(0, _NGRP // 2, super_step, 0)
        wait_idx(src0, dst0, is0)
        plsc.subcore_barrier()
        for z in range(_ZROWS // 128):
            r0 = sid * _ZROWS + z * 128
            pltpu.sync_copy(acc_sh.at[pl.ds(r0, 128)], rows0)
            pltpu.sync_copy(rows0, out_p.at[cid, pl.ds(r0, 128)])

    return pl.kernel(body,
                     out_type=(jax.ShapeDtypeStruct((2, _ROWS_SH, _D),
                                                    jnp.float32),),
                     mesh=_SC_MESH, scratch_types=scratch)


def _make_deg():
    scratch = [
        pltpu.VMEM((_GRP, _CHUNK), jnp.int32),
        pltpu.VMEM((_CHUNK, _D), jnp.float32),
        pltpu.VMEM_SHARED((_ROWS_SH, _D), jnp.float32),
        pltpu.SemaphoreType.DMA,
    ]

    def body(dst3, z128, ones128, deg_p, dst_v, buf_v, deg_sh, sem):
        cid = lax.axis_index("c")
        sid = lax.axis_index("s")
        wid = cid * 16 + sid
        pltpu.sync_copy(z128, buf_v)
        for z in range(_ZROWS // 128):
            pltpu.sync_copy(buf_v, deg_sh.at[pl.ds(sid * _ZROWS + z * 128, 128)])
        pltpu.sync_copy(ones128, buf_v)
        plsc.subcore_barrier()

        def group(g, carry):
            pltpu.sync_copy(dst3.at[wid, pl.ds(g * _GRP, _GRP)], dst_v)
            for j in range(_GRP):
                pltpu.sync_copy(buf_v, deg_sh.at[dst_v.at[j]], add=True)
            return carry

        lax.fori_loop(0, _NGRP, group, 0)
        plsc.subcore_barrier()
        for z in range(_ZROWS // 128):
            r0 = sid * _ZROWS + z * 128
            pltpu.sync_copy(deg_sh.at[pl.ds(r0, 128)], buf_v)
            pltpu.sync_copy(buf_v, deg_p.at[cid, pl.ds(r0, 128)])

    return pl.kernel(body,
                     out_type=(jax.ShapeDtypeStruct((2, _ROWS_SH, _D),
                                                    jnp.float32),),
                     mesh=_SC_MESH, scratch_types=scratch)


_agg = _make_agg()
_deg = _make_deg()


def _wspec():
    return pl.BlockSpec((_D, _D), lambda i: (0, 0))


def _bspec():
    return pl.BlockSpec((_D,), lambda i: (0,))


def _hspec():
    return pl.BlockSpec((_BLK, _D), lambda i: (i, 0))


def _stage_in(feats, W_in, b_in, W1, b1):
    def body(f, win, bin_, w1, b1_, h_o, m_o):
        h = _gelu(_dot(f[...], win[...]) + bin_[...])
        h_o[...] = h
        m_o[...] = _gelu(_dot(h, w1[...]) + b1_[...])

    return pl.pallas_call(
        body,
        grid=(_GRID,),
        in_specs=[_hspec(), _wspec(), _bspec(), _wspec(), _bspec()],
        out_specs=[_hspec(), _hspec()],
        out_shape=[jax.ShapeDtypeStruct((_N, _D), jnp.float32)] * 2,
    )(feats, W_in, b_in, W1, b1)


_PSPEC = pl.BlockSpec((2, _BLK, _D), lambda i: (0, i, 0))
_DSPEC = pl.BlockSpec((2, _BLK, _D), lambda i: (0, i, 0))


def _stage_mid(h, p, dp, W2, b2, W3, b3, Wl, bl, W1n, b1n):
    def body(h_r, p_r, d_r, w2, b2_, w3, b3_, wl, bl_, w1n, b1n_, h_o, m_o):
        d = d_r[0, :, :1] + d_r[1, :, :1]
        agg = (p_r[0] + p_r[1]) * (1.0 / jnp.maximum(d, 1.0))
        h0 = h_r[...]
        t = _gelu(_dot(agg, w2[...]) + b2_[...] + _dot(h0, w3[...]) + b3_[...])
        hn = _dot(t, wl[...]) + bl_[...] + h0
        h_o[...] = hn
        m_o[...] = _gelu(_dot(hn, w1n[...]) + b1n_[...])

    return pl.pallas_call(
        body,
        grid=(_GRID,),
        in_specs=[_hspec(), _PSPEC, _DSPEC, _wspec(), _bspec(), _wspec(),
                  _bspec(), _wspec(), _bspec(), _wspec(), _bspec()],
        out_specs=[_hspec(), _hspec()],
        out_shape=[jax.ShapeDtypeStruct((_N, _D), jnp.float32)] * 2,
    )(h, p, dp, W2, b2, W3, b3, Wl, bl, W1n, b1n)


def _stage_out(h, p, dp, W2, b2, W3, b3, Wl, bl, W_out, b_out):
    def body(h_r, p_r, d_r, w2, b2_, w3, b3_, wl, bl_, wo, bo_, o_o):
        d = d_r[0, :, :1] + d_r[1, :, :1]
        agg = (p_r[0] + p_r[1]) * (1.0 / jnp.maximum(d, 1.0))
        h0 = h_r[...]
        t = _gelu(_dot(agg, w2[...]) + b2_[...] + _dot(h0, w3[...]) + b3_[...])
        hn = _dot(t, wl[...]) + bl_[...] + h0
        o_o[...] = _dot(hn, wo[...]) + bo_[...]

    return pl.pallas_call(
        body,
        grid=(_GRID,),
        in_specs=[_hspec(), _PSPEC, _DSPEC, _wspec(), _bspec(), _wspec(),
                  _bspec(), _wspec(), _bspec(), _wspec(), _bspec()],
        out_specs=_hspec(),
        out_shape=jax.ShapeDtypeStruct((_N, _D), jnp.float32),
    )(h, p, dp, W2, b2, W3, b3, Wl, bl, W_out, b_out)


def kernel(feats, edge_index, W_in, b_in,
           W1_0, b1_0, W2_0, b2_0, W3_0, b3_0, Wl_0, bl_0,
           W1_1, b1_1, W2_1, b2_1, W3_1, b3_1, Wl_1, bl_1,
           W_out, b_out):
    src = edge_index[0]
    dst = edge_index[1]
    pad = _EPAD - _E
    src3 = jnp.concatenate(
        [src, jnp.zeros((pad,), jnp.int32)]).reshape(_NTILE, _NCHUNK, _CHUNK)
    src3 = jnp.concatenate(
        [src3, jnp.zeros((_NTILE, _GRP, _CHUNK), jnp.int32)], axis=1)
    dst3 = jnp.concatenate(
        [dst, jnp.full((pad,), _N, jnp.int32)]).reshape(_NTILE, _NCHUNK, _CHUNK)
    dst3 = jnp.concatenate(
        [dst3, jnp.full((_NTILE, _GRP, _CHUNK), _N, jnp.int32)], axis=1)
    z128 = jnp.zeros((128, _D), jnp.float32)
    ones128 = jnp.ones((_CHUNK, _D), jnp.float32)

    (dp,) = _deg(dst3, z128, ones128)
    h, m = _stage_in(feats, W_in, b_in, W1_0, b1_0)
    (p0,) = _agg(src3, dst3, z128, m)
    h, m = _stage_mid(h, p0, dp, W2_0, b2_0, W3_0, b3_0, Wl_0, bl_0,
                      W1_1, b1_1)
    (p1,) = _agg(src3, dst3, z128, m)
    out = _stage_out(h, p1, dp, W2_1, b2_1, W3_1, b3_1, Wl_1, bl_1,
                     W_out, b_out)
    return out
